# baseline (speedup 1.0000x reference)
import jax
import jax.numpy as jnp
from jax.experimental import pallas as pl
from jax.experimental.pallas import tpu as pltpu

_NB = 2
_LOG2E = 1.4426950408889634
_LN2 = 0.6931471805599453


def _softplus_cheap(x, mv):
    a = jnp.abs(x)
    m = jnp.maximum(x, 0.0)
    t = jnp.exp2(a * (-_LOG2E))
    return m + jnp.log2(1.0 + t) * _LN2 + mv


def _body(mv_ref, x_ref, o_ref):
    j = pl.program_id(1)

    @pl.when(j == 0)
    def _transform():
        mv = jnp.logaddexp(mv_ref[0], 0.0)
        o_ref[...] = _softplus_cheap(x_ref[...], mv)

    @pl.when(j == 1)
    def _mixed():
        mv = jnp.logaddexp(mv_ref[0], 0.0)
        x = x_ref[...]
        ch = jax.lax.broadcasted_iota(jnp.int32, x.shape, 3)
        o_ref[...] = jnp.where(ch < 64, _softplus_cheap(x, mv), x)

    @pl.when(j == 2)
    def _copy():
        o_ref[...] = x_ref[...]


def kernel(input_, _min_value):
    n, c, h, w = input_.shape
    xt = jnp.transpose(input_, (0, 2, 3, 1))
    mv = jnp.asarray(_min_value, jnp.float32).reshape(1)
    out = pl.pallas_call(
        _body,
        grid=(n // _NB, 3),
        in_specs=[
            pl.BlockSpec(memory_space=pltpu.SMEM),
            pl.BlockSpec((_NB, h, w, 128), lambda i, j: (i, 0, 0, j)),
        ],
        out_specs=pl.BlockSpec((_NB, h, w, 128), lambda i, j: (i, 0, 0, j)),
        out_shape=jax.ShapeDtypeStruct((n, h, w, c), input_.dtype),
        compiler_params=pltpu.CompilerParams(
            dimension_semantics=("parallel", "parallel"),
        ),
    )(mv, xt)
    return jnp.transpose(out, (0, 3, 1, 2))


# speedup vs baseline: 1.2333x; 1.2333x over previous
import jax
import jax.numpy as jnp
from jax.experimental import pallas as pl
from jax.experimental.pallas import tpu as pltpu

_NB = 2
_LOG2E = 1.4426950408889634
_LN2 = 0.6931471805599453


def _body(mv_ref, x_ref, o_ref):
    x = x_ref[...]
    mv = jnp.logaddexp(mv_ref[0], 0.0)
    a = jnp.abs(x)
    m = jnp.maximum(x, 0.0)
    t = jnp.exp2(a * (-_LOG2E))
    sp = m + jnp.log2(1.0 + t) * _LN2 + mv
    ch = jax.lax.broadcasted_iota(jnp.int32, x.shape, 3)
    o_ref[...] = jnp.where(ch < 192, sp, x)


def kernel(input_, _min_value):
    n, c, h, w = input_.shape
    xt = jnp.transpose(input_, (0, 2, 3, 1))
    mv = jnp.asarray(_min_value, jnp.float32).reshape(1)
    out = pl.pallas_call(
        _body,
        grid=(n // _NB,),
        in_specs=[
            pl.BlockSpec(memory_space=pltpu.SMEM),
            pl.BlockSpec((_NB, h, w, c), lambda i: (i, 0, 0, 0)),
        ],
        out_specs=pl.BlockSpec((_NB, h, w, c), lambda i: (i, 0, 0, 0)),
        out_shape=jax.ShapeDtypeStruct((n, h, w, c), input_.dtype),
        compiler_params=pltpu.CompilerParams(
            dimension_semantics=("parallel",),
        ),
    )(mv, xt)
    return jnp.transpose(out, (0, 3, 1, 2))
